# baseline (speedup 1.0000x reference)
import functools

import jax
import jax.numpy as jnp
from jax.experimental import pallas as pl
from jax.experimental.pallas import tpu as pltpu

_THRESHOLD = 0.2
_TEMPERATURE = 1.0


def _fused_body(B, E, R, nsteps,
                x_ref, w_ref, b_ref, v_ref, vt_ref,
                o_ref, coeffs_ref, mask_ref,
                tok_ref, nsq_ref):
    i = pl.program_id(0)

    o_ref[...] = (
        jnp.dot(x_ref[...].astype(jnp.bfloat16), w_ref[...],
                preferred_element_type=jnp.float32)
        + b_ref[...]
    )

    @pl.when(i == 0)
    def _cache_tok():
        tok_ref[...] = x_ref[:B, :]
        nsq_ref[...] = jnp.zeros_like(nsq_ref)

    @pl.when(i < E)
    def _expert_column():
        tok = tok_ref[...]
        proj_e = jnp.dot(tok, v_ref[0],
                         preferred_element_type=jnp.float32)
        recon_e = jnp.dot(proj_e, vt_ref[0],
                          preferred_element_type=jnp.float32)
        resid_e = tok - recon_e
        col = jnp.sum(resid_e * resid_e, axis=1, keepdims=True)
        lane = jax.lax.broadcasted_iota(jnp.int32, (B, E), 1)
        nsq_ref[...] += jnp.where(lane == i, col, 0.0)

    @pl.when(i == nsteps - 1)
    def _gate():
        nsq = nsq_ref[...]
        logits = -jnp.sqrt(nsq + 1e-12)
        mean = jnp.mean(logits, axis=1, keepdims=True)
        var = jnp.sum((logits - mean) ** 2, axis=1, keepdims=True) / (E - 1)
        std = jnp.sqrt(var) + 1e-06
        z = (logits - mean) / std / _TEMPERATURE
        zmax = jnp.max(z, axis=1, keepdims=True)
        ez = jnp.exp(z - zmax)
        coeffs_ref[...] = ez / jnp.sum(ez, axis=1, keepdims=True)
        mask_ref[...] = (coeffs_ref[...] > _THRESHOLD).astype(jnp.int8)


@functools.partial(jax.jit, static_argnames=("bm",))
def _run(x, v, s, W, b, bm=3152):
    SEQ, B, D = x.shape
    E, _, R = v.shape
    M = SEQ * B
    xm = x.reshape(M, D)
    Wt = W.T.astype(jnp.bfloat16)
    b2 = b.reshape(1, D)
    vt = v.transpose(0, 2, 1)
    grid = M // bm
    ecap = E - 1
    out, coeffs, mask_i8 = pl.pallas_call(
        functools.partial(_fused_body, B, E, R, grid),
        grid=(grid,),
        in_specs=[
            pl.BlockSpec((bm, D), lambda i: (i, 0)),
            pl.BlockSpec((D, D), lambda i: (0, 0)),
            pl.BlockSpec((1, D), lambda i: (0, 0)),
            pl.BlockSpec((1, D, R), lambda i: (jnp.minimum(i, ecap), 0, 0)),
            pl.BlockSpec((1, R, D), lambda i: (jnp.minimum(i, ecap), 0, 0)),
        ],
        out_specs=[
            pl.BlockSpec((bm, D), lambda i: (i, 0)),
            pl.BlockSpec((B, E), lambda i: (0, 0)),
            pl.BlockSpec((B, E), lambda i: (0, 0)),
        ],
        out_shape=[
            jax.ShapeDtypeStruct((M, D), jnp.float32),
            jax.ShapeDtypeStruct((B, E), jnp.float32),
            jax.ShapeDtypeStruct((B, E), jnp.int8),
        ],
        scratch_shapes=[
            pltpu.VMEM((B, D), jnp.float32),
            pltpu.VMEM((B, E), jnp.float32),
        ],
        compiler_params=pltpu.CompilerParams(
            dimension_semantics=("arbitrary",)),
    )(xm, Wt, b2, v, vt)
    return out.reshape(SEQ, B, D), coeffs, mask_i8.astype(jnp.bool_)


def kernel(x, v, s, W, b, bsz):
    return _run(x, v, s, W, b)


# speedup vs baseline: 1.0483x; 1.0483x over previous
import functools

import jax
import jax.numpy as jnp
from jax.experimental import pallas as pl
from jax.experimental.pallas import tpu as pltpu

_THRESHOLD = 0.2
_TEMPERATURE = 1.0


def _fused_body(B, E, R, x_ref, w_ref, b_ref, vflat_ref, sel_ref,
                o_ref, coeffs_ref, mask_ref):
    o_ref[...] = (
        jnp.dot(x_ref[...].astype(jnp.bfloat16), w_ref[...],
                preferred_element_type=jnp.float32)
        + b_ref[...]
    )

    @pl.when(pl.program_id(0) == 0)
    def _routing():
        tok = x_ref[:B, :]
        proj = jnp.dot(tok, vflat_ref[...],
                       preferred_element_type=jnp.float32)
        projsq = jnp.dot(proj * proj, sel_ref[...],
                         preferred_element_type=jnp.float32)
        tok2 = jnp.sum(tok * tok, axis=1, keepdims=True)
        nsq = tok2 - projsq
        logits = -jnp.sqrt(nsq + 1e-12)
        mean = jnp.mean(logits, axis=1, keepdims=True)
        var = jnp.sum((logits - mean) ** 2, axis=1, keepdims=True) / (E - 1)
        std = jnp.sqrt(var) + 1e-06
        z = (logits - mean) / std / _TEMPERATURE
        zmax = jnp.max(z, axis=1, keepdims=True)
        ez = jnp.exp(z - zmax)
        coeffs_ref[...] = ez / jnp.sum(ez, axis=1, keepdims=True)
        mask_ref[...] = (coeffs_ref[...] > _THRESHOLD).astype(jnp.int8)


@functools.partial(jax.jit, static_argnames=("bm",))
def _run(x, v, s, W, b, bm=3152):
    SEQ, B, D = x.shape
    E, _, R = v.shape
    M = SEQ * B
    xm = x.reshape(M, D)
    Wt = W.T.astype(jnp.bfloat16)
    b2 = b.reshape(1, D)
    vflat = v.transpose(1, 0, 2).reshape(D, E * R)
    sel = (jax.lax.broadcasted_iota(jnp.int32, (E * R, E), 0) // R
           == jax.lax.broadcasted_iota(jnp.int32, (E * R, E), 1)
           ).astype(jnp.float32)
    grid = M // bm
    out, coeffs, mask_i8 = pl.pallas_call(
        functools.partial(_fused_body, B, E, R),
        grid=(grid,),
        in_specs=[
            pl.BlockSpec((bm, D), lambda i: (i, 0)),
            pl.BlockSpec((D, D), lambda i: (0, 0)),
            pl.BlockSpec((1, D), lambda i: (0, 0)),
            pl.BlockSpec((D, E * R), lambda i: (0, 0)),
            pl.BlockSpec((E * R, E), lambda i: (0, 0)),
        ],
        out_specs=[
            pl.BlockSpec((bm, D), lambda i: (i, 0)),
            pl.BlockSpec((B, E), lambda i: (0, 0)),
            pl.BlockSpec((B, E), lambda i: (0, 0)),
        ],
        out_shape=[
            jax.ShapeDtypeStruct((M, D), jnp.float32),
            jax.ShapeDtypeStruct((B, E), jnp.float32),
            jax.ShapeDtypeStruct((B, E), jnp.int8),
        ],
        compiler_params=pltpu.CompilerParams(
            dimension_semantics=("arbitrary",)),
    )(xm, Wt, b2, vflat, sel)
    return out.reshape(SEQ, B, D), coeffs, mask_i8.astype(jnp.bool_)


def kernel(x, v, s, W, b, bsz):
    return _run(x, v, s, W, b)
